# baseline (speedup 1.0000x reference)
import jax
import jax.numpy as jnp
from jax.experimental import pallas as pl
from jax.experimental.pallas import tpu as pltpu

D = 1024
N_NEIGH = 26
NPAD = 32
CH = 512
NC1 = 2 * D // CH
NC2 = D // CH
NSEM = 3 * NC1 + NC2


def _decode(v):
    q729 = jnp.floor((v + 0.5) * (1.0 / 729.0))
    q27 = jnp.floor((v + 0.5) * (1.0 / 27.0))
    return q729, q27 - 27.0 * q729, v - 27.0 * q27


def _masks(nidx_ref, cell_ref):
    f32 = jnp.float32
    idxf = nidx_ref[...].astype(f32)
    cellf = cell_ref[...].astype(f32)
    nx, ny, nz = _decode(idxf)
    cx, cy, cz = _decode(cellf)
    d2 = (nx - cx) ** 2 + (ny - cy) ** 2 + (nz - cz) ** 2
    lane = jax.lax.broadcasted_iota(jnp.int32, (1, NPAD), 1)
    valid = (lane < N_NEIGH).astype(f32)
    lm = (d2 <= 3.5).astype(f32) * valid
    fm = ((d2 > 3.5) & (d2 <= 20.5)).astype(f32) * valid
    dm = (d2 > 20.5).astype(f32) * valid
    return lm, fm, dm, valid


def _body(cs_ref, ns_ref, nidx_ref, cell_ref, wf1_hbm, wl_hbm, wd_hbm,
          wf2_hbm, wg_ref, bl_ref, bf1_ref, bf2_ref, bd_ref, bg_ref,
          out_state_ref, out_ew_ref, wf1_v, wl_v, wd_v, wf2_v, sems):
    f32 = jnp.float32

    copies = []
    k = 0
    for c in range(NC1):
        rows = pl.ds(c * CH, CH)
        for src, dst in ((wf1_hbm, wf1_v), (wl_hbm, wl_v), (wd_hbm, wd_v)):
            cp = pltpu.make_async_copy(src.at[rows, :], dst.at[rows, :],
                                       sems.at[k])
            cp.start()
            copies.append(cp)
            k += 1
    f2_copies = []
    for c in range(NC2):
        rows = pl.ds(c * CH, CH)
        cp = pltpu.make_async_copy(wf2_hbm.at[rows, :], wf2_v.at[rows, :],
                                   sems.at[k])
        cp.start()
        f2_copies.append(cp)
        k += 1

    lm, fm, dm, valid = _masks(nidx_ref, cell_ref)
    lc = jnp.sum(lm, axis=1, keepdims=True)
    fc = jnp.sum(fm, axis=1, keepdims=True)
    dc = jnp.sum(dm, axis=1, keepdims=True)
    coeff = jnp.concatenate([
        lm / jnp.maximum(lc, 1.0),
        fm / jnp.maximum(fc, 1.0),
        dm / jnp.maximum(dc, 1.0),
        valid * (1.0 / N_NEIGH),
    ], axis=0)
    means = jnp.dot(coeff, ns_ref[...], preferred_element_type=f32)
    cs = cs_ref[...]
    xg = jnp.concatenate([cs, means[3:4, :]], axis=1)
    glog = jnp.dot(xg, wg_ref[...], preferred_element_type=f32)

    xs = [jnp.concatenate([cs, means[r:r + 1, :]], axis=1)
          for r in (1, 0, 2)]

    u = [jnp.zeros((1, D), f32) for _ in range(3)]
    k = 0
    for c in range(NC1):
        cols = slice(c * CH, (c + 1) * CH)
        for e, buf in enumerate((wf1_v, wl_v, wd_v)):
            copies[k].wait()
            u[e] = u[e] + jnp.dot(xs[e][:, cols], buf[cols, :],
                                  preferred_element_type=f32)
            k += 1

    h1 = jnp.tanh(u[0] + bf1_ref[...])
    u2 = jnp.zeros((1, D), f32)
    for c in range(NC2):
        cols = slice(c * CH, (c + 1) * CH)
        f2_copies[c].wait()
        u2 = u2 + jnp.dot(h1[:, cols], wf2_v[cols, :],
                          preferred_element_type=f32)

    local_out = jnp.tanh(u[1] + bl_ref[...])
    local_out = jnp.where(lc > 0.0, local_out, 0.0)
    func_out = jnp.tanh(u2 + bf2_ref[...]) + cs
    func_out = jnp.where(fc > 0.0, func_out, 0.0)
    dist_out = jnp.tanh(u[2] + bd_ref[...])
    dist_out = jnp.where(dc > 0.0, dist_out, 0.0)

    g = jnp.pad(glog, ((0, 0), (0, 128 - 3))) + bg_ref[...]
    lane128 = jax.lax.broadcasted_iota(jnp.int32, (1, 128), 1)
    m3 = lane128 < 3
    gmax = jnp.max(jnp.where(m3, g, -jnp.inf), axis=1, keepdims=True)
    e = jnp.where(m3, jnp.exp(g - gmax), 0.0)
    w = e / jnp.sum(e, axis=1, keepdims=True)
    out_ew_ref[...] = w
    out_state_ref[...] = (w[0:1, 0:1] * local_out
                          + w[0:1, 1:2] * func_out
                          + w[0:1, 2:3] * dist_out)


def kernel(current_state, neighbor_states, cell_idx, neighbor_indices,
           W_local, b_local, W_f1, b_f1, W_f2, b_f2, W_dist, b_dist,
           W_gate, b_gate):
    f32 = jnp.float32
    cs2 = current_state.reshape(1, D)
    ns_p = jnp.pad(neighbor_states, ((0, NPAD - N_NEIGH), (0, 0)))
    nidx = jnp.pad(jnp.asarray(neighbor_indices, jnp.int32),
                   (0, NPAD - N_NEIGH)).reshape(1, NPAD)
    cell = jnp.asarray(cell_idx, jnp.int32).reshape(1, 1)
    bg_p = jnp.pad(b_gate, (0, 128 - 3)).reshape(1, 128)

    vmem = pl.BlockSpec(memory_space=pltpu.MemorySpace.VMEM)
    hbm = pl.BlockSpec(memory_space=pltpu.MemorySpace.HBM)

    out_state, out_ew = pl.pallas_call(
        _body,
        in_specs=[vmem, vmem, vmem, vmem,
                  hbm, hbm, hbm, hbm,
                  vmem,
                  vmem, vmem, vmem, vmem, vmem],
        out_specs=[vmem, vmem],
        out_shape=[jax.ShapeDtypeStruct((1, D), f32),
                   jax.ShapeDtypeStruct((1, 128), f32)],
        scratch_shapes=[pltpu.VMEM((2 * D, D), f32),
                        pltpu.VMEM((2 * D, D), f32),
                        pltpu.VMEM((2 * D, D), f32),
                        pltpu.VMEM((D, D), f32),
                        pltpu.SemaphoreType.DMA((NSEM,))],
    )(cs2, ns_p, nidx, cell, W_f1, W_local, W_dist, W_f2, W_gate,
      b_local.reshape(1, D), b_f1.reshape(1, D), b_f2.reshape(1, D),
      b_dist.reshape(1, D), bg_p)

    return out_state.reshape(D), out_ew[0, :3]
